# baseline (speedup 1.0000x reference)
import jax
import jax.numpy as jnp
from jax import lax
from jax.experimental import pallas as pl
from jax.experimental.pallas import tpu as pltpu

NUM_CLASS = 8
EMBED = 256
B = 1024
N = 32
NUM_MODES = 20
D_IN = 100

_BLK_B = 1024
_GRID = B * N // _BLK_B
_BLK_A = B // _GRID
_ROWS2 = _BLK_A * NUM_MODES
_K2 = NUM_CLASS * NUM_MODES + _BLK_A + 1


def _body(neis_ref, nlbl_ref, Wn_ref, b_nei_ref,
          obs_ref, slbl_ref, midx_ref, Wo_ref, b_obs_ref,
          Wm_ref, modes_ref, b_mode_ref,
          nei_out_ref, x_out_ref, m2_ref, obs_row_ref):
    i = pl.program_id(0)

    ident = (lax.broadcasted_iota(jnp.int32, (D_IN, D_IN), 0) ==
             lax.broadcasted_iota(jnp.int32, (D_IN, D_IN), 1)
             ).astype(jnp.bfloat16)

    @pl.when(i == 0)
    def _():
        m2_ref[...] = lax.dot_general(
            modes_ref[...].astype(jnp.bfloat16),
            Wm_ref[:, EMBED:].astype(jnp.bfloat16),
            (((1,), (1,)), ((), ())),
            preferred_element_type=jnp.float32)
        obs_row_ref[...] = lax.dot_general(
            obs_ref[...], ident, (((0,), (0,)), ((), ())),
            preferred_element_type=jnp.float32,
            ).astype(jnp.bfloat16)

    v = neis_ref[...].astype(jnp.float32)
    t = jnp.where(v >= 0, 1.0 / (v + 0.0001), 1.0 / (v - 0.0001))
    p = lax.dot_general(t.astype(jnp.bfloat16), Wn_ref[...].astype(jnp.bfloat16),
                        (((1,), (1,)), ((), ())),
                        preferred_element_type=jnp.float32)
    nlbl = nlbl_ref[0, :, :]
    acc = jnp.zeros((_BLK_B, EMBED), jnp.float32)
    for c in range(NUM_CLASS + 1):
        feat = p[:, c * EMBED:(c + 1) * EMBED] + b_nei_ref[c, :][None, :]
        acc = jnp.where(nlbl == c, feat, acc)
    nei_out_ref[...] = acc

    ob = obs_row_ref[pl.ds(i * _BLK_A, _BLK_A), :]
    y_all = lax.dot_general(ob,
                            Wo_ref[...].astype(jnp.bfloat16),
                            (((1,), (1,)), ((), ())),
                            preferred_element_type=jnp.float32)
    slbl = slbl_ref[0, :, :]
    x = jnp.zeros((_BLK_A, EMBED), jnp.float32)
    for c in range(NUM_CLASS):
        feat = y_all[:, c * EMBED:(c + 1) * EMBED] + b_obs_ref[c, :][None, :]
        x = jnp.where(slbl == c, feat, x)
    xw1 = lax.dot_general(x.astype(jnp.bfloat16),
                          Wm_ref[:, :EMBED].astype(jnp.bfloat16),
                          (((1,), (1,)), ((), ())),
                          preferred_element_type=jnp.float32)

    midx = midx_ref[0, :, :]
    j = lax.broadcasted_iota(jnp.int32, (_ROWS2, _K2), 1)
    rid = lax.broadcasted_iota(jnp.int32, (_ROWS2, 1), 0) % _BLK_A
    sel = ((j == midx) | (j == rid + NUM_CLASS * NUM_MODES) | (j == _K2 - 1))
    a_mat = sel.astype(jnp.bfloat16)
    w_big = jnp.concatenate(
        [m2_ref[...], xw1, b_mode_ref[...]], axis=0).astype(jnp.bfloat16)
    res = jnp.dot(a_mat, w_big,
                  preferred_element_type=jnp.float32)
    x_out_ref[...] = res.reshape(NUM_MODES, _BLK_A, EMBED)


def kernel(obs, neis, self_labels, nei_labels, modes,
           W_obs, b_obs, W_nei, b_nei, W_mode, b_mode):
    obs_p = obs.astype(jnp.bfloat16).transpose(1, 2, 0).reshape(D_IN, B)
    neis_p = neis.astype(jnp.bfloat16).reshape(B * N, D_IN)
    Wo = W_obs.reshape(NUM_CLASS * EMBED, D_IN)
    Wn = W_nei.reshape((NUM_CLASS + 1) * EMBED, D_IN)
    modes_flat = modes.reshape(NUM_CLASS * NUM_MODES, EMBED)
    b_mode2 = b_mode.reshape(1, EMBED)
    slbl = self_labels.reshape(_GRID, _BLK_A, 1)
    nlbl = nei_labels.reshape(_GRID, _BLK_B, 1)
    midx = (self_labels.reshape(_GRID, 1, _BLK_A) * NUM_MODES
            + jnp.arange(NUM_MODES, dtype=self_labels.dtype).reshape(1, NUM_MODES, 1)
            ).reshape(_GRID, _ROWS2, 1)

    nei_out, x_out = pl.pallas_call(
        _body,
        grid=(_GRID,),
        in_specs=[
            pl.BlockSpec((_BLK_B, D_IN), lambda i: (i, 0)),
            pl.BlockSpec((1, _BLK_B, 1), lambda i: (i, 0, 0)),
            pl.BlockSpec(((NUM_CLASS + 1) * EMBED, D_IN), lambda i: (0, 0)),
            pl.BlockSpec((NUM_CLASS + 1, EMBED), lambda i: (0, 0)),
            pl.BlockSpec((D_IN, B), lambda i: (0, 0)),
            pl.BlockSpec((1, _BLK_A, 1), lambda i: (i, 0, 0)),
            pl.BlockSpec((1, _ROWS2, 1), lambda i: (i, 0, 0)),
            pl.BlockSpec((NUM_CLASS * EMBED, D_IN), lambda i: (0, 0)),
            pl.BlockSpec((NUM_CLASS, EMBED), lambda i: (0, 0)),
            pl.BlockSpec((EMBED, 2 * EMBED), lambda i: (0, 0)),
            pl.BlockSpec((NUM_CLASS * NUM_MODES, EMBED), lambda i: (0, 0)),
            pl.BlockSpec((1, EMBED), lambda i: (0, 0)),
        ],
        out_specs=[
            pl.BlockSpec((_BLK_B, EMBED), lambda i: (i, 0)),
            pl.BlockSpec((NUM_MODES, _BLK_A, EMBED), lambda i: (0, i, 0)),
        ],
        out_shape=[
            jax.ShapeDtypeStruct((B * N, EMBED), jnp.float32),
            jax.ShapeDtypeStruct((NUM_MODES, B, EMBED), jnp.float32),
        ],
        scratch_shapes=[pltpu.VMEM((NUM_CLASS * NUM_MODES, EMBED), jnp.float32),
                        pltpu.VMEM((B, D_IN), jnp.bfloat16)],
    )(neis_p, nlbl, Wn, b_nei, obs_p, slbl, midx, Wo, b_obs,
      W_mode, modes_flat, b_mode2)

    return (x_out.transpose(1, 0, 2), nei_out.reshape(B, N, EMBED))


# speedup vs baseline: 1.0108x; 1.0108x over previous
import jax
import jax.numpy as jnp
from jax import lax
from jax.experimental import pallas as pl
from jax.experimental.pallas import tpu as pltpu

NUM_CLASS = 8
EMBED = 256
B = 1024
N = 32
NUM_MODES = 20
D_IN = 100

_BLK_B = 2048
_GRID = B * N // _BLK_B
_BLK_A = B // _GRID
_ROWS2 = _BLK_A * NUM_MODES
_K2 = NUM_CLASS * NUM_MODES + _BLK_A + 1


def _body(neis_ref, nlbl_ref, Wn_ref, b_nei_ref,
          obs_ref, slbl_ref, midx_ref, Wo_ref, b_obs_ref,
          Wm_ref, modes_ref, b_mode_ref,
          nei_out_ref, x_out_ref, m2_ref, obs_row_ref):
    i = pl.program_id(0)

    ident = (lax.broadcasted_iota(jnp.int32, (D_IN, D_IN), 0) ==
             lax.broadcasted_iota(jnp.int32, (D_IN, D_IN), 1)
             ).astype(jnp.bfloat16)

    @pl.when(i == 0)
    def _():
        m2_ref[...] = lax.dot_general(
            modes_ref[...].astype(jnp.bfloat16),
            Wm_ref[:, EMBED:].astype(jnp.bfloat16),
            (((1,), (1,)), ((), ())),
            preferred_element_type=jnp.float32)
        obs_row_ref[...] = lax.dot_general(
            obs_ref[...], ident, (((0,), (0,)), ((), ())),
            preferred_element_type=jnp.float32,
            ).astype(jnp.bfloat16)

    v = neis_ref[...].astype(jnp.float32)
    t = jnp.where(v >= 0, 1.0 / (v + 0.0001), 1.0 / (v - 0.0001))
    p = lax.dot_general(t.astype(jnp.bfloat16), Wn_ref[...].astype(jnp.bfloat16),
                        (((1,), (1,)), ((), ())),
                        preferred_element_type=jnp.float32)
    nlbl = nlbl_ref[0, :, :]
    acc = jnp.zeros((_BLK_B, EMBED), jnp.float32)
    for c in range(NUM_CLASS + 1):
        feat = p[:, c * EMBED:(c + 1) * EMBED] + b_nei_ref[c, :][None, :]
        acc = jnp.where(nlbl == c, feat, acc)
    nei_out_ref[...] = acc

    ob = obs_row_ref[pl.ds(i * _BLK_A, _BLK_A), :]
    y_all = lax.dot_general(ob,
                            Wo_ref[...].astype(jnp.bfloat16),
                            (((1,), (1,)), ((), ())),
                            preferred_element_type=jnp.float32)
    slbl = slbl_ref[0, :, :]
    x = jnp.zeros((_BLK_A, EMBED), jnp.float32)
    for c in range(NUM_CLASS):
        feat = y_all[:, c * EMBED:(c + 1) * EMBED] + b_obs_ref[c, :][None, :]
        x = jnp.where(slbl == c, feat, x)
    xw1 = lax.dot_general(x.astype(jnp.bfloat16),
                          Wm_ref[:, :EMBED].astype(jnp.bfloat16),
                          (((1,), (1,)), ((), ())),
                          preferred_element_type=jnp.float32)

    midx = midx_ref[0, :, :]
    j = lax.broadcasted_iota(jnp.int32, (_ROWS2, _K2), 1)
    rid = lax.broadcasted_iota(jnp.int32, (_ROWS2, 1), 0) % _BLK_A
    sel = ((j == midx) | (j == rid + NUM_CLASS * NUM_MODES) | (j == _K2 - 1))
    a_mat = sel.astype(jnp.bfloat16)
    w_big = jnp.concatenate(
        [m2_ref[...], xw1, b_mode_ref[...]], axis=0).astype(jnp.bfloat16)
    res = jnp.dot(a_mat, w_big,
                  preferred_element_type=jnp.float32)
    x_out_ref[...] = res.reshape(NUM_MODES, _BLK_A, EMBED)


def kernel(obs, neis, self_labels, nei_labels, modes,
           W_obs, b_obs, W_nei, b_nei, W_mode, b_mode):
    obs_p = obs.astype(jnp.bfloat16).transpose(1, 2, 0).reshape(D_IN, B)
    neis_p = neis.astype(jnp.bfloat16).reshape(B * N, D_IN)
    Wo = W_obs.reshape(NUM_CLASS * EMBED, D_IN)
    Wn = W_nei.reshape((NUM_CLASS + 1) * EMBED, D_IN)
    modes_flat = modes.reshape(NUM_CLASS * NUM_MODES, EMBED)
    b_mode2 = b_mode.reshape(1, EMBED)
    slbl = self_labels.reshape(_GRID, _BLK_A, 1)
    nlbl = nei_labels.reshape(_GRID, _BLK_B, 1)
    midx = (self_labels.reshape(_GRID, 1, _BLK_A) * NUM_MODES
            + jnp.arange(NUM_MODES, dtype=self_labels.dtype).reshape(1, NUM_MODES, 1)
            ).reshape(_GRID, _ROWS2, 1)

    nei_out, x_out = pl.pallas_call(
        _body,
        grid=(_GRID,),
        in_specs=[
            pl.BlockSpec((_BLK_B, D_IN), lambda i: (i, 0)),
            pl.BlockSpec((1, _BLK_B, 1), lambda i: (i, 0, 0)),
            pl.BlockSpec(((NUM_CLASS + 1) * EMBED, D_IN), lambda i: (0, 0)),
            pl.BlockSpec((NUM_CLASS + 1, EMBED), lambda i: (0, 0)),
            pl.BlockSpec((D_IN, B), lambda i: (0, 0)),
            pl.BlockSpec((1, _BLK_A, 1), lambda i: (i, 0, 0)),
            pl.BlockSpec((1, _ROWS2, 1), lambda i: (i, 0, 0)),
            pl.BlockSpec((NUM_CLASS * EMBED, D_IN), lambda i: (0, 0)),
            pl.BlockSpec((NUM_CLASS, EMBED), lambda i: (0, 0)),
            pl.BlockSpec((EMBED, 2 * EMBED), lambda i: (0, 0)),
            pl.BlockSpec((NUM_CLASS * NUM_MODES, EMBED), lambda i: (0, 0)),
            pl.BlockSpec((1, EMBED), lambda i: (0, 0)),
        ],
        out_specs=[
            pl.BlockSpec((_BLK_B, EMBED), lambda i: (i, 0)),
            pl.BlockSpec((NUM_MODES, _BLK_A, EMBED), lambda i: (0, i, 0)),
        ],
        out_shape=[
            jax.ShapeDtypeStruct((B * N, EMBED), jnp.float32),
            jax.ShapeDtypeStruct((NUM_MODES, B, EMBED), jnp.float32),
        ],
        scratch_shapes=[pltpu.VMEM((NUM_CLASS * NUM_MODES, EMBED), jnp.float32),
                        pltpu.VMEM((B, D_IN), jnp.bfloat16)],
    )(neis_p, nlbl, Wn, b_nei, obs_p, slbl, midx, Wo, b_obs,
      W_mode, modes_flat, b_mode2)

    return (x_out.transpose(1, 0, 2), nei_out.reshape(B, N, EMBED))


# speedup vs baseline: 1.0433x; 1.0321x over previous
import jax
import jax.numpy as jnp
from jax import lax
from jax.experimental import pallas as pl
from jax.experimental.pallas import tpu as pltpu

NUM_CLASS = 8
EMBED = 256
B = 1024
N = 32
NUM_MODES = 20
D_IN = 100

_BLK_B = 4096
_GRID = B * N // _BLK_B
_BLK_A = B // _GRID
_ROWS2 = _BLK_A * NUM_MODES
_K2 = NUM_CLASS * NUM_MODES + _BLK_A + 1


def _body(neis_ref, nlbl_ref, Wn_ref, b_nei_ref,
          obs_ref, slbl_ref, midx_ref, Wo_ref, b_obs_ref,
          Wm_ref, modes_ref, b_mode_ref,
          nei_out_ref, x_out_ref, m2_ref, obs_row_ref):
    i = pl.program_id(0)

    ident = (lax.broadcasted_iota(jnp.int32, (D_IN, D_IN), 0) ==
             lax.broadcasted_iota(jnp.int32, (D_IN, D_IN), 1)
             ).astype(jnp.bfloat16)

    @pl.when(i == 0)
    def _():
        m2_ref[...] = lax.dot_general(
            modes_ref[...].astype(jnp.bfloat16),
            Wm_ref[:, EMBED:].astype(jnp.bfloat16),
            (((1,), (1,)), ((), ())),
            preferred_element_type=jnp.float32)
        obs_row_ref[...] = lax.dot_general(
            obs_ref[...], ident, (((0,), (0,)), ((), ())),
            preferred_element_type=jnp.float32,
            ).astype(jnp.bfloat16)

    v = neis_ref[...].astype(jnp.float32)
    t = jnp.where(v >= 0, 1.0 / (v + 0.0001), 1.0 / (v - 0.0001))
    p = lax.dot_general(t.astype(jnp.bfloat16), Wn_ref[...].astype(jnp.bfloat16),
                        (((1,), (1,)), ((), ())),
                        preferred_element_type=jnp.float32)
    nlbl = nlbl_ref[0, :, :]
    acc = jnp.zeros((_BLK_B, EMBED), jnp.float32)
    for c in range(NUM_CLASS + 1):
        feat = p[:, c * EMBED:(c + 1) * EMBED] + b_nei_ref[c, :][None, :]
        acc = jnp.where(nlbl == c, feat, acc)
    nei_out_ref[...] = acc

    ob = obs_row_ref[pl.ds(i * _BLK_A, _BLK_A), :]
    y_all = lax.dot_general(ob,
                            Wo_ref[...].astype(jnp.bfloat16),
                            (((1,), (1,)), ((), ())),
                            preferred_element_type=jnp.float32)
    slbl = slbl_ref[0, :, :]
    x = jnp.zeros((_BLK_A, EMBED), jnp.float32)
    for c in range(NUM_CLASS):
        feat = y_all[:, c * EMBED:(c + 1) * EMBED] + b_obs_ref[c, :][None, :]
        x = jnp.where(slbl == c, feat, x)
    xw1 = lax.dot_general(x.astype(jnp.bfloat16),
                          Wm_ref[:, :EMBED].astype(jnp.bfloat16),
                          (((1,), (1,)), ((), ())),
                          preferred_element_type=jnp.float32)

    midx = midx_ref[0, :, :]
    j = lax.broadcasted_iota(jnp.int32, (_ROWS2, _K2), 1)
    rid = lax.broadcasted_iota(jnp.int32, (_ROWS2, 1), 0) % _BLK_A
    sel = ((j == midx) | (j == rid + NUM_CLASS * NUM_MODES) | (j == _K2 - 1))
    a_mat = sel.astype(jnp.bfloat16)
    w_big = jnp.concatenate(
        [m2_ref[...], xw1, b_mode_ref[...]], axis=0).astype(jnp.bfloat16)
    res = jnp.dot(a_mat, w_big,
                  preferred_element_type=jnp.float32)
    x_out_ref[...] = res.reshape(NUM_MODES, _BLK_A, EMBED)


def kernel(obs, neis, self_labels, nei_labels, modes,
           W_obs, b_obs, W_nei, b_nei, W_mode, b_mode):
    obs_p = obs.astype(jnp.bfloat16).transpose(1, 2, 0).reshape(D_IN, B)
    neis_p = neis.astype(jnp.bfloat16).reshape(B * N, D_IN)
    Wo = W_obs.reshape(NUM_CLASS * EMBED, D_IN)
    Wn = W_nei.reshape((NUM_CLASS + 1) * EMBED, D_IN)
    modes_flat = modes.reshape(NUM_CLASS * NUM_MODES, EMBED)
    b_mode2 = b_mode.reshape(1, EMBED)
    slbl = self_labels.reshape(_GRID, _BLK_A, 1)
    nlbl = nei_labels.reshape(_GRID, _BLK_B, 1)
    midx = (self_labels.reshape(_GRID, 1, _BLK_A) * NUM_MODES
            + jnp.arange(NUM_MODES, dtype=self_labels.dtype).reshape(1, NUM_MODES, 1)
            ).reshape(_GRID, _ROWS2, 1)

    nei_out, x_out = pl.pallas_call(
        _body,
        grid=(_GRID,),
        in_specs=[
            pl.BlockSpec((_BLK_B, D_IN), lambda i: (i, 0)),
            pl.BlockSpec((1, _BLK_B, 1), lambda i: (i, 0, 0)),
            pl.BlockSpec(((NUM_CLASS + 1) * EMBED, D_IN), lambda i: (0, 0)),
            pl.BlockSpec((NUM_CLASS + 1, EMBED), lambda i: (0, 0)),
            pl.BlockSpec((D_IN, B), lambda i: (0, 0)),
            pl.BlockSpec((1, _BLK_A, 1), lambda i: (i, 0, 0)),
            pl.BlockSpec((1, _ROWS2, 1), lambda i: (i, 0, 0)),
            pl.BlockSpec((NUM_CLASS * EMBED, D_IN), lambda i: (0, 0)),
            pl.BlockSpec((NUM_CLASS, EMBED), lambda i: (0, 0)),
            pl.BlockSpec((EMBED, 2 * EMBED), lambda i: (0, 0)),
            pl.BlockSpec((NUM_CLASS * NUM_MODES, EMBED), lambda i: (0, 0)),
            pl.BlockSpec((1, EMBED), lambda i: (0, 0)),
        ],
        out_specs=[
            pl.BlockSpec((_BLK_B, EMBED), lambda i: (i, 0)),
            pl.BlockSpec((NUM_MODES, _BLK_A, EMBED), lambda i: (0, i, 0)),
        ],
        out_shape=[
            jax.ShapeDtypeStruct((B * N, EMBED), jnp.float32),
            jax.ShapeDtypeStruct((NUM_MODES, B, EMBED), jnp.float32),
        ],
        scratch_shapes=[pltpu.VMEM((NUM_CLASS * NUM_MODES, EMBED), jnp.float32),
                        pltpu.VMEM((B, D_IN), jnp.bfloat16)],
    )(neis_p, nlbl, Wn, b_nei, obs_p, slbl, midx, Wo, b_obs,
      W_mode, modes_flat, b_mode2)

    return (x_out.transpose(1, 0, 2), nei_out.reshape(B, N, EMBED))


# speedup vs baseline: 1.0900x; 1.0448x over previous
import jax
import jax.numpy as jnp
from jax import lax
from jax.experimental import pallas as pl
from jax.experimental.pallas import tpu as pltpu

NUM_CLASS = 8
EMBED = 256
B = 1024
N = 32
NUM_MODES = 20
D_IN = 100

_BLK_B = 4096
_GRID = B * N // _BLK_B
_BLK_A = B // _GRID
_ROWS2 = _BLK_A * NUM_MODES
_K2 = NUM_CLASS * NUM_MODES + _BLK_A + 1


def _body(neis_ref, nlbl_ref, Wn_ref, b_nei_ref,
          obs_ref, slbl_ref, midx_ref, Wo_ref, b_obs_ref,
          Wm_ref, modes_ref, b_mode_ref,
          nei_out_ref, x_out_ref, m2_ref, obs_row_ref):
    i = pl.program_id(0)

    ident = (lax.broadcasted_iota(jnp.int32, (D_IN, D_IN), 0) ==
             lax.broadcasted_iota(jnp.int32, (D_IN, D_IN), 1)
             ).astype(jnp.bfloat16)

    @pl.when(i == 0)
    def _():
        m2_ref[...] = lax.dot_general(
            modes_ref[...].astype(jnp.bfloat16),
            Wm_ref[:, EMBED:].astype(jnp.bfloat16),
            (((1,), (1,)), ((), ())),
            preferred_element_type=jnp.float32)
        obs_row_ref[...] = lax.dot_general(
            obs_ref[...], ident, (((0,), (0,)), ((), ())),
            preferred_element_type=jnp.float32,
            ).astype(jnp.bfloat16)

    v = neis_ref[...].astype(jnp.float32)
    t = jnp.where(v >= 0, 1.0 / (v + 0.0001), 1.0 / (v - 0.0001))
    p = lax.dot_general(t.astype(jnp.bfloat16), Wn_ref[...].astype(jnp.bfloat16),
                        (((1,), (1,)), ((), ())),
                        preferred_element_type=jnp.float32)
    nlbl = nlbl_ref[0, :, :]
    acc = p[:, :EMBED] + b_nei_ref[0, :][None, :]
    for c in range(1, NUM_CLASS + 1):
        feat = p[:, c * EMBED:(c + 1) * EMBED] + b_nei_ref[c, :][None, :]
        acc = jnp.where(nlbl == c, feat, acc)
    nei_out_ref[...] = acc

    ob = obs_row_ref[pl.ds(i * _BLK_A, _BLK_A), :]
    y_all = lax.dot_general(ob,
                            Wo_ref[...].astype(jnp.bfloat16),
                            (((1,), (1,)), ((), ())),
                            preferred_element_type=jnp.float32)
    slbl = slbl_ref[0, :, :]
    x = y_all[:, :EMBED] + b_obs_ref[0, :][None, :]
    for c in range(1, NUM_CLASS):
        feat = y_all[:, c * EMBED:(c + 1) * EMBED] + b_obs_ref[c, :][None, :]
        x = jnp.where(slbl == c, feat, x)
    xw1 = lax.dot_general(x.astype(jnp.bfloat16),
                          Wm_ref[:, :EMBED].astype(jnp.bfloat16),
                          (((1,), (1,)), ((), ())),
                          preferred_element_type=jnp.float32)

    midx = midx_ref[0, :, :]
    j = lax.broadcasted_iota(jnp.int32, (_ROWS2, _K2), 1)
    rid = lax.broadcasted_iota(jnp.int32, (_ROWS2, 1), 0) % _BLK_A
    sel = ((j == midx) | (j == rid + NUM_CLASS * NUM_MODES) | (j == _K2 - 1))
    a_mat = sel.astype(jnp.bfloat16)
    w_big = jnp.concatenate(
        [m2_ref[...], xw1, b_mode_ref[...]], axis=0).astype(jnp.bfloat16)
    res = jnp.dot(a_mat, w_big,
                  preferred_element_type=jnp.float32)
    x_out_ref[...] = res.reshape(NUM_MODES, _BLK_A, EMBED)


def kernel(obs, neis, self_labels, nei_labels, modes,
           W_obs, b_obs, W_nei, b_nei, W_mode, b_mode):
    obs_p = obs.astype(jnp.bfloat16).transpose(1, 2, 0).reshape(D_IN, B)
    neis_p = neis.astype(jnp.bfloat16).reshape(B * N, D_IN)
    Wo = W_obs.reshape(NUM_CLASS * EMBED, D_IN)
    Wn = W_nei.reshape((NUM_CLASS + 1) * EMBED, D_IN)
    modes_flat = modes.reshape(NUM_CLASS * NUM_MODES, EMBED)
    b_mode2 = b_mode.reshape(1, EMBED)
    slbl = self_labels.reshape(_GRID, _BLK_A, 1)
    nlbl = nei_labels.reshape(_GRID, _BLK_B, 1)
    midx = (self_labels.reshape(_GRID, 1, _BLK_A) * NUM_MODES
            + jnp.arange(NUM_MODES, dtype=self_labels.dtype).reshape(1, NUM_MODES, 1)
            ).reshape(_GRID, _ROWS2, 1)

    nei_out, x_out = pl.pallas_call(
        _body,
        grid=(_GRID,),
        in_specs=[
            pl.BlockSpec((_BLK_B, D_IN), lambda i: (i, 0)),
            pl.BlockSpec((1, _BLK_B, 1), lambda i: (i, 0, 0)),
            pl.BlockSpec(((NUM_CLASS + 1) * EMBED, D_IN), lambda i: (0, 0)),
            pl.BlockSpec((NUM_CLASS + 1, EMBED), lambda i: (0, 0)),
            pl.BlockSpec((D_IN, B), lambda i: (0, 0)),
            pl.BlockSpec((1, _BLK_A, 1), lambda i: (i, 0, 0)),
            pl.BlockSpec((1, _ROWS2, 1), lambda i: (i, 0, 0)),
            pl.BlockSpec((NUM_CLASS * EMBED, D_IN), lambda i: (0, 0)),
            pl.BlockSpec((NUM_CLASS, EMBED), lambda i: (0, 0)),
            pl.BlockSpec((EMBED, 2 * EMBED), lambda i: (0, 0)),
            pl.BlockSpec((NUM_CLASS * NUM_MODES, EMBED), lambda i: (0, 0)),
            pl.BlockSpec((1, EMBED), lambda i: (0, 0)),
        ],
        out_specs=[
            pl.BlockSpec((_BLK_B, EMBED), lambda i: (i, 0)),
            pl.BlockSpec((NUM_MODES, _BLK_A, EMBED), lambda i: (0, i, 0)),
        ],
        out_shape=[
            jax.ShapeDtypeStruct((B * N, EMBED), jnp.float32),
            jax.ShapeDtypeStruct((NUM_MODES, B, EMBED), jnp.float32),
        ],
        scratch_shapes=[pltpu.VMEM((NUM_CLASS * NUM_MODES, EMBED), jnp.float32),
                        pltpu.VMEM((B, D_IN), jnp.bfloat16)],
    )(neis_p, nlbl, Wn, b_nei, obs_p, slbl, midx, Wo, b_obs,
      W_mode, modes_flat, b_mode2)

    return (x_out.transpose(1, 0, 2), nei_out.reshape(B, N, EMBED))
